# baseline (speedup 1.0000x reference)
import functools

import jax
import jax.numpy as jnp
from jax.experimental import pallas as pl
from jax.experimental.pallas import tpu as pltpu

_D = ((0, -1), (0, 1))
_KH = {(0, 0): 1, (0, -1): 3, (1, 0): 2, (1, 1): 0}

_DN = (((0,), (0,)), ((), ()))


def _shift(x, off):
    m = x.shape[-1]
    s = off % m
    if s == 0:
        return x
    return jnp.concatenate([x[:, s:], x[:, :s]], axis=1)


def _l0_kernel(z_ref, w_ref, g_ref, b_ref, o_ref, *, n, taps):
    cout = o_ref.shape[0]
    wv = w_ref[...]
    z = z_ref[...]
    s = jnp.zeros((cout, 1), jnp.float32)
    ss = jnp.zeros((cout, 1), jnp.float32)
    ys = []
    for t in range(taps):
        y = jax.lax.dot_general(wv[:, t, :], z, _DN,
                                preferred_element_type=jnp.float32)
        ys.append(y)
        s = s + jnp.sum(y, axis=1, keepdims=True)
        ss = ss + jnp.sum(y * y, axis=1, keepdims=True)
    cnt = float(taps * n)
    mean = s / cnt
    var = ss / cnt - mean * mean
    inv = jax.lax.rsqrt(var + 1e-5)
    scale = g_ref[...] * inv
    shift = b_ref[...] - mean * scale
    for t in range(taps):
        o_ref[:, t * n:(t + 1) * n] = jnp.maximum(
            ys[t] * scale + shift, 0.0).astype(o_ref.dtype)


def _taps_in_kernel(x, n, h, w):
    m = n * h * w
    col = jax.lax.broadcasted_iota(jnp.int32, (1, m), 1)
    r = col // (w * n)
    c = (col // n) % w
    taps = {}
    for d in (-1, 0, 1):
        for e in (-1, 0, 1):
            ok = (r + d >= 0) & (r + d < h) & (c + e >= 0) & (c + e < w)
            taps[(d, e)] = jnp.where(ok, _shift(x, (d * w + e) * n), 0)
    return taps


def _phase_acc(wv, taps, p):
    ph, pw = divmod(p, 2)
    acc = None
    for jd in range(2):
        d = _D[ph][jd]
        for je in range(2):
            e = _D[pw][je]
            q = _KH[(ph, d)] * 4 + _KH[(pw, e)]
            t = jax.lax.dot_general(wv[:, q, :], taps[(d, e)], _DN,
                                    preferred_element_type=jnp.float32)
            acc = t if acc is None else acc + t
    return acc


def _up_bn_relu_kernel(x_ref, w_ref, g_ref, b_ref, o_ref, *, n, h, w):
    cp = o_ref.shape[1]
    m = n * h * w
    taps = _taps_in_kernel(x_ref[...], n, h, w)
    wv = w_ref[...]
    s = jnp.zeros((cp, 1), jnp.float32)
    ss = jnp.zeros((cp, 1), jnp.float32)
    ys = []
    for p in range(4):
        y = _phase_acc(wv, taps, p)
        ys.append(y)
        s = s + jnp.sum(y, axis=1, keepdims=True)
        ss = ss + jnp.sum(y * y, axis=1, keepdims=True)
    cnt = float(4 * m)
    mean = s / cnt
    var = ss / cnt - mean * mean
    inv = jax.lax.rsqrt(var + 1e-5)
    scale = g_ref[...] * inv
    shift = b_ref[...] - mean * scale
    for p in range(4):
        o_ref[p] = jnp.maximum(ys[p] * scale + shift, 0.0).astype(o_ref.dtype)


def _up_bias_tanh_kernel(x_ref, w_ref, b_ref, o_ref, *, n, h, w):
    taps = _taps_in_kernel(x_ref[...], n, h, w)
    wv = w_ref[...]
    for p in range(4):
        y = _phase_acc(wv, taps, p)
        o_ref[p] = jnp.tanh(y + b_ref[...]).astype(o_ref.dtype)


def _pack_w(w_t, cp):
    cin, cout, k, _ = w_t.shape
    wp = jnp.transpose(w_t, (0, 2, 3, 1)).reshape(cin, k * k, cout)
    wp = wp.astype(jnp.bfloat16)
    if cp != cout:
        wp = jnp.pad(wp, ((0, 0), (0, 0), (0, cp - cout)))
    return wp


_VMEM = 60000 * 1024


def _l0(z, w_t, gamma, beta):
    cin, cout, k, _ = w_t.shape
    n = z.shape[0]
    taps = k * k
    wp = _pack_w(w_t, cout)
    g = gamma.astype(jnp.float32).reshape(cout, 1)
    b = beta.astype(jnp.float32).reshape(cout, 1)
    zt = jnp.transpose(z).astype(jnp.bfloat16)
    return pl.pallas_call(
        functools.partial(_l0_kernel, n=n, taps=taps),
        out_shape=jax.ShapeDtypeStruct((cout, taps * n), jnp.bfloat16),
        grid=(1,),
        in_specs=[pl.BlockSpec((cin, n), lambda i: (0, 0)),
                  pl.BlockSpec((cin, taps, cout), lambda i: (0, 0, 0)),
                  pl.BlockSpec((cout, 1), lambda i: (0, 0)),
                  pl.BlockSpec((cout, 1), lambda i: (0, 0))],
        out_specs=pl.BlockSpec((cout, taps * n), lambda i: (0, 0)),
        compiler_params=pltpu.CompilerParams(
            dimension_semantics=("arbitrary",), vmem_limit_bytes=_VMEM),
    )(zt, wp, g, b)


def _reassemble(y, cout, n, h, w):
    cp = y.shape[1]
    yr = y.reshape(2, 2, cp, h, w, n)[:, :, :cout]
    yr = jnp.transpose(yr, (2, 3, 0, 4, 1, 5))
    return yr.reshape(cout, 4 * h * w * n)


def _up(x_cm, w_t, gamma, beta, n, h, w):
    cin, cout = w_t.shape[0], w_t.shape[1]
    cp = ((cout + 7) // 8) * 8
    m = n * h * w
    wp = _pack_w(w_t, cp)
    g = jnp.pad(gamma.astype(jnp.float32), (0, cp - cout)).reshape(cp, 1)
    b = jnp.pad(beta.astype(jnp.float32), (0, cp - cout)).reshape(cp, 1)
    y = pl.pallas_call(
        functools.partial(_up_bn_relu_kernel, n=n, h=h, w=w),
        out_shape=jax.ShapeDtypeStruct((4, cp, m), jnp.bfloat16),
        grid=(1,),
        in_specs=[pl.BlockSpec((cin, m), lambda i: (0, 0)),
                  pl.BlockSpec((cin, 16, cp), lambda i: (0, 0, 0)),
                  pl.BlockSpec((cp, 1), lambda i: (0, 0)),
                  pl.BlockSpec((cp, 1), lambda i: (0, 0))],
        out_specs=pl.BlockSpec((4, cp, m), lambda i: (0, 0, 0)),
        compiler_params=pltpu.CompilerParams(
            dimension_semantics=("arbitrary",), vmem_limit_bytes=_VMEM),
    )(x_cm, wp, g, b)
    return _reassemble(y, cout, n, h, w)


def _final(x_cm, w_t, bias, n, h, w):
    cin, cout = w_t.shape[0], w_t.shape[1]
    cp = ((cout + 7) // 8) * 8
    m = n * h * w
    wp = _pack_w(w_t, cp)
    b = jnp.pad(bias.astype(jnp.float32), (0, cp - cout)).reshape(cp, 1)
    y = pl.pallas_call(
        functools.partial(_up_bias_tanh_kernel, n=n, h=h, w=w),
        out_shape=jax.ShapeDtypeStruct((4, cp, m), jnp.float32),
        grid=(1,),
        in_specs=[pl.BlockSpec((cin, m), lambda i: (0, 0)),
                  pl.BlockSpec((cin, 16, cp), lambda i: (0, 0, 0)),
                  pl.BlockSpec((cp, 1), lambda i: (0, 0))],
        out_specs=pl.BlockSpec((4, cp, m), lambda i: (0, 0, 0)),
        compiler_params=pltpu.CompilerParams(
            dimension_semantics=("arbitrary",), vmem_limit_bytes=_VMEM),
    )(x_cm, wp, b)
    yr = y.reshape(2, 2, cp, h, w, n)[:, :, :cout]
    img = jnp.transpose(yr, (5, 2, 3, 0, 4, 1))
    return img.reshape(n, cout, 2 * h, 2 * w)


def kernel(w0, gamma0, beta0, w1, gamma1, beta1, w2, gamma2, beta2,
           w3, gamma3, beta3, w4, bias4, x):
    n = x.shape[0]
    z = x.reshape(n, x.shape[1])
    h0 = _l0(z, w0, gamma0, beta0)
    h1 = _up(h0, w1, gamma1, beta1, n, 4, 4)
    h2 = _up(h1, w2, gamma2, beta2, n, 8, 8)
    h3 = _up(h2, w3, gamma3, beta3, n, 16, 16)
    return _final(h3, w4, bias4, n, 32, 32)


# speedup vs baseline: 1.1214x; 1.1214x over previous
import functools

import jax
import jax.numpy as jnp
from jax.experimental import pallas as pl
from jax.experimental.pallas import tpu as pltpu

_D = ((0, -1), (0, 1))
_KH = {(0, 0): 1, (0, -1): 3, (1, 0): 2, (1, 1): 0}


def _shift(x, off):
    m = x.shape[-1]
    s = off % m
    if s == 0:
        return x
    return jnp.concatenate([x[:, s:], x[:, :s]], axis=1)


def _l0_kernel(z_ref, w_ref, g_ref, b_ref, o_ref, *, n, taps):
    cout = o_ref.shape[0]
    s = jnp.zeros((cout, 1), jnp.float32)
    ss = jnp.zeros((cout, 1), jnp.float32)
    ys = []
    for t in range(taps):
        y = jnp.dot(w_ref[t], z_ref[...], preferred_element_type=jnp.float32)
        ys.append(y)
        s = s + jnp.sum(y, axis=1, keepdims=True)
        ss = ss + jnp.sum(y * y, axis=1, keepdims=True)
    cnt = float(taps * n)
    mean = s / cnt
    var = ss / cnt - mean * mean
    inv = jax.lax.rsqrt(var + 1e-5)
    scale = g_ref[...] * inv
    shift = b_ref[...] - mean * scale
    for t in range(taps):
        o_ref[:, t * n:(t + 1) * n] = jnp.maximum(
            ys[t] * scale + shift, 0.0).astype(o_ref.dtype)


def _taps_in_kernel(x, n, h, w):
    m = n * h * w
    col = jax.lax.broadcasted_iota(jnp.int32, (1, m), 1)
    r = col // (w * n)
    c = (col // n) % w
    taps = {}
    for d in (-1, 0, 1):
        for e in (-1, 0, 1):
            ok = (r + d >= 0) & (r + d < h) & (c + e >= 0) & (c + e < w)
            taps[(d, e)] = jnp.where(ok, _shift(x, (d * w + e) * n), 0)
    return taps


def _phase_acc(w_ref, taps, p):
    ph, pw = divmod(p, 2)
    acc = None
    for jd in range(2):
        d = _D[ph][jd]
        for je in range(2):
            e = _D[pw][je]
            t = jnp.dot(w_ref[p * 4 + jd * 2 + je], taps[(d, e)],
                        preferred_element_type=jnp.float32)
            acc = t if acc is None else acc + t
    return acc


def _up_bn_relu_kernel(x_ref, w_ref, g_ref, b_ref, o_ref, *, n, h, w):
    cp = o_ref.shape[1]
    m = n * h * w
    taps = _taps_in_kernel(x_ref[...], n, h, w)
    s = jnp.zeros((cp, 1), jnp.float32)
    ss = jnp.zeros((cp, 1), jnp.float32)
    ys = []
    for p in range(4):
        y = _phase_acc(w_ref, taps, p)
        ys.append(y)
        s = s + jnp.sum(y, axis=1, keepdims=True)
        ss = ss + jnp.sum(y * y, axis=1, keepdims=True)
    cnt = float(4 * m)
    mean = s / cnt
    var = ss / cnt - mean * mean
    inv = jax.lax.rsqrt(var + 1e-5)
    scale = g_ref[...] * inv
    shift = b_ref[...] - mean * scale
    for p in range(4):
        o_ref[p] = jnp.maximum(ys[p] * scale + shift, 0.0).astype(o_ref.dtype)


def _up_bias_tanh_kernel(x_ref, w_ref, b_ref, o_ref, *, n, h, w):
    taps = _taps_in_kernel(x_ref[...], n, h, w)
    for p in range(4):
        y = _phase_acc(w_ref, taps, p)
        o_ref[p] = jnp.tanh(y + b_ref[...]).astype(o_ref.dtype)


def _pack_w(w_t, cp):
    cin, cout, k, _ = w_t.shape
    wt = jnp.transpose(w_t, (2, 3, 1, 0)).astype(jnp.bfloat16)
    blocks = []
    for ph in range(2):
        for pw in range(2):
            for jd in range(2):
                d = _D[ph][jd]
                kh = _KH[(ph, d)]
                for je in range(2):
                    e = _D[pw][je]
                    kw = _KH[(pw, e)]
                    b = wt[kh, kw]
                    if cp != cout:
                        b = jnp.pad(b, ((0, cp - cout), (0, 0)))
                    blocks.append(b)
    return jnp.stack(blocks)


_VMEM = 48 * 1024 * 1024


def _l0(z, w_t, gamma, beta):
    cin, cout, k, _ = w_t.shape
    n = z.shape[0]
    taps = k * k
    wp = jnp.transpose(w_t, (2, 3, 1, 0)).reshape(taps, cout, cin)
    wp = wp.astype(jnp.bfloat16)
    g = gamma.astype(jnp.float32).reshape(cout, 1)
    b = beta.astype(jnp.float32).reshape(cout, 1)
    zt = jnp.transpose(z).astype(jnp.bfloat16)
    return pl.pallas_call(
        functools.partial(_l0_kernel, n=n, taps=taps),
        out_shape=jax.ShapeDtypeStruct((cout, taps * n), jnp.bfloat16),
        grid=(1,),
        in_specs=[pl.BlockSpec((cin, n), lambda i: (0, 0)),
                  pl.BlockSpec((taps, cout, cin), lambda i: (0, 0, 0)),
                  pl.BlockSpec((cout, 1), lambda i: (0, 0)),
                  pl.BlockSpec((cout, 1), lambda i: (0, 0))],
        out_specs=pl.BlockSpec((cout, taps * n), lambda i: (0, 0)),
        compiler_params=pltpu.CompilerParams(
            dimension_semantics=("arbitrary",), vmem_limit_bytes=_VMEM),
    )(zt, wp, g, b)


def _reassemble(y, cout, n, h, w):
    cp = y.shape[1]
    yr = y.reshape(2, 2, cp, h, w, n)[:, :, :cout]
    yr = jnp.transpose(yr, (2, 3, 0, 4, 1, 5))
    return yr.reshape(cout, 4 * h * w * n)


def _up(x_cm, w_t, gamma, beta, n, h, w):
    cin, cout = w_t.shape[0], w_t.shape[1]
    cp = ((cout + 7) // 8) * 8
    m = n * h * w
    wp = _pack_w(w_t, cp)
    g = jnp.pad(gamma.astype(jnp.float32), (0, cp - cout)).reshape(cp, 1)
    b = jnp.pad(beta.astype(jnp.float32), (0, cp - cout)).reshape(cp, 1)
    y = pl.pallas_call(
        functools.partial(_up_bn_relu_kernel, n=n, h=h, w=w),
        out_shape=jax.ShapeDtypeStruct((4, cp, m), jnp.bfloat16),
        grid=(1,),
        in_specs=[pl.BlockSpec((cin, m), lambda i: (0, 0)),
                  pl.BlockSpec((16, cp, cin), lambda i: (0, 0, 0)),
                  pl.BlockSpec((cp, 1), lambda i: (0, 0)),
                  pl.BlockSpec((cp, 1), lambda i: (0, 0))],
        out_specs=pl.BlockSpec((4, cp, m), lambda i: (0, 0, 0)),
        compiler_params=pltpu.CompilerParams(
            dimension_semantics=("arbitrary",), vmem_limit_bytes=_VMEM),
    )(x_cm, wp, g, b)
    return _reassemble(y, cout, n, h, w)


def _final(x_cm, w_t, bias, n, h, w):
    cin, cout = w_t.shape[0], w_t.shape[1]
    cp = ((cout + 7) // 8) * 8
    m = n * h * w
    wp = _pack_w(w_t, cp)
    b = jnp.pad(bias.astype(jnp.float32), (0, cp - cout)).reshape(cp, 1)
    y = pl.pallas_call(
        functools.partial(_up_bias_tanh_kernel, n=n, h=h, w=w),
        out_shape=jax.ShapeDtypeStruct((4, cp, m), jnp.float32),
        grid=(1,),
        in_specs=[pl.BlockSpec((cin, m), lambda i: (0, 0)),
                  pl.BlockSpec((16, cp, cin), lambda i: (0, 0, 0)),
                  pl.BlockSpec((cp, 1), lambda i: (0, 0))],
        out_specs=pl.BlockSpec((4, cp, m), lambda i: (0, 0, 0)),
        compiler_params=pltpu.CompilerParams(
            dimension_semantics=("arbitrary",), vmem_limit_bytes=_VMEM),
    )(x_cm, wp, b)
    yr = y.reshape(2, 2, cp, h, w, n)[:, :, :cout]
    img = jnp.transpose(yr, (5, 2, 3, 0, 4, 1))
    return img.reshape(n, cout, 2 * h, 2 * w)


def kernel(w0, gamma0, beta0, w1, gamma1, beta1, w2, gamma2, beta2,
           w3, gamma3, beta3, w4, bias4, x):
    n = x.shape[0]
    z = x.reshape(n, x.shape[1])
    h0 = _l0(z, w0, gamma0, beta0)
    h1 = _up(h0, w1, gamma1, beta1, n, 4, 4)
    h2 = _up(h1, w2, gamma2, beta2, n, 8, 8)
    h3 = _up(h2, w3, gamma3, beta3, n, 16, 16)
    return _final(h3, w4, bias4, n, 32, 32)
